# baseline (speedup 1.0000x reference)
import functools
import math

import jax
import jax.numpy as jnp
from jax import lax
from jax.experimental import pallas as pl
from jax.experimental.pallas import tpu as pltpu
from jax.experimental.pallas import tpu_sc as plsc

N = 10000
D = 128
E = 320000
ALPHA = 0.1
MIX_B = math.log1p(0.5 / 3.0)

NC = 2
NS = 16
NW = NC * NS
L = 16

EPW = E // NW
EPC = E // NC
RPS = N // NS
RCHUNK = 80
C = 40
CHUNKS = EPW // C
R = 8
BN = 1000

_mesh = plsc.VectorSubcoreMesh(core_axis_name="c", subcore_axis_name="s")


@functools.partial(
    pl.kernel,
    out_type=jax.ShapeDtypeStruct((NW * N,), jnp.float32),
    mesh=_mesh,
    scratch_types=[
        pltpu.VMEM((EPW,), jnp.int32),
        pltpu.VMEM((N,), jnp.float32),
    ],
    compiler_params=pltpu.CompilerParams(needs_layout_passes=False),
)
def _deg_kernel(edges_hbm, deg_out, dstbuf, degbuf):
    cid = lax.axis_index("c")
    sid = lax.axis_index("s")
    wid = sid * NC + cid

    zeros = jnp.zeros((L,), jnp.float32)
    ones = jnp.ones((L,), jnp.float32)

    def _zero(i):
        degbuf[pl.ds(i * L, L)] = zeros

    pl.loop(0, N // L)(_zero)

    pltpu.sync_copy(edges_hbm.at[pl.ds(E + wid * EPW, EPW)], dstbuf)

    def _count(i):
        idx = dstbuf[pl.ds(i * L, L)]
        plsc.addupdate_scatter(degbuf, [idx], ones)

    pl.loop(0, EPW // L)(_count)

    pltpu.sync_copy(degbuf, deg_out.at[pl.ds(wid * N, N)])


def _scale_body(deg_ref, f_ref, g_ref):
    deg = jnp.sum(deg_ref[0], axis=0) + 1.0
    dinv = lax.rsqrt(deg)
    g_ref[...] = f_ref[...] * dinv[:, None]


_scale_kernel = pl.pallas_call(
    _scale_body,
    grid=(N // BN,),
    in_specs=[
        pl.BlockSpec((1, NW, BN), lambda j: (j, 0, 0)),
        pl.BlockSpec((BN, D), lambda j: (j, 0)),
    ],
    out_specs=pl.BlockSpec((BN, D), lambda j: (j, 0)),
    out_shape=jax.ShapeDtypeStruct((N, D), jnp.float32),
)


@functools.partial(
    pl.kernel,
    out_type=jax.ShapeDtypeStruct((NC, N, D), jnp.float32),
    mesh=_mesh,
    scratch_types=[
        pltpu.VMEM_SHARED((N, D), jnp.float32),
        pltpu.VMEM((2 * R, C), jnp.int32),
        pltpu.VMEM((2 * R, C), jnp.int32),
        pltpu.VMEM((R * C, D), jnp.float32),
        pltpu.SemaphoreType.DMA((R,)),
        pltpu.SemaphoreType.DMA((R,)),
        pltpu.SemaphoreType.DMA((R,)),
        pltpu.SemaphoreType.DMA,
    ],
)
def _agg_kernel(edges_hbm, g_hbm, s_out, acc, srcb, dstb,
                rows, gsems, ssems, dsems, asem):
    cid = lax.axis_index("c")
    sid = lax.axis_index("s")
    wid = cid * NS + sid

    zv = jnp.zeros((L,), jnp.float32)

    def _zrow(r):
        def _zcol(j):
            rows[r, pl.ds(j * L, L)] = zv

        pl.loop(0, D // L)(_zcol)

    pl.loop(0, C)(_zrow)

    def _zero(c):
        pltpu.sync_copy(rows.at[pl.ds(0, C)],
                        acc.at[pl.ds(c * C, C)])

    pl.loop(sid, N // C, step=NS)(_zero)

    def _idx_start(t):
        s = t % (2 * R)
        off = wid * EPW + t * C
        pltpu.async_copy(edges_hbm.at[pl.ds(off, C)], srcb.at[s],
                         ssems.at[t % R])
        pltpu.async_copy(edges_hbm.at[pl.ds(E + off, C)], dstb.at[s],
                         dsems.at[t % R])

    def _idx_wait(t):
        s = t % (2 * R)
        pltpu.make_async_copy(edges_hbm.at[pl.ds(0, C)], srcb.at[s],
                              ssems.at[t % R]).wait()
        pltpu.make_async_copy(edges_hbm.at[pl.ds(0, C)], dstb.at[s],
                              dsems.at[t % R]).wait()

    def _gather_start(t):
        pltpu.async_copy(g_hbm.at[srcb.at[t % (2 * R)]],
                         rows.at[pl.ds((t % R) * C, C)], gsems.at[t % R])

    def _gather_wait(t):
        pltpu.make_async_copy(g_hbm.at[pl.ds(0, C)],
                              rows.at[pl.ds((t % R) * C, C)],
                              gsems.at[t % R]).wait()

    def _scatter_wait(t):
        pltpu.make_async_copy(rows.at[pl.ds((t % R) * C, C)],
                              acc.at[dstb.at[t % (2 * R)]], asem).wait()

    for t in range(R):
        _idx_start(t)
    plsc.subcore_barrier()
    for t in range(R - 1):
        _idx_wait(t)
        _gather_start(t)

    def _step(t):
        _gather_wait(t)

        @pl.when(t > 0)
        def _():
            _scatter_wait(t - 1)

        @pl.when(t + R - 1 < CHUNKS)
        def _():
            _idx_wait(t + R - 1)
            _gather_start(t + R - 1)

        @pl.when(t + R < CHUNKS)
        def _():
            _idx_start(t + R)

        pltpu.async_copy(rows.at[pl.ds((t % R) * C, C)],
                         acc.at[dstb.at[t % (2 * R)]], asem, add=True)

    pl.loop(0, CHUNKS)(_step)

    _scatter_wait(CHUNKS - 1)
    plsc.subcore_barrier()

    def _dump(c):
        pltpu.sync_copy(acc.at[pl.ds(c * RCHUNK, RCHUNK)],
                        s_out.at[cid, pl.ds(c * RCHUNK, RCHUNK)])

    pl.loop(sid, N // RCHUNK, step=NS)(_dump)


def _final_body(deg_ref, s_ref, g_ref, h_ref, w_ref, o_ref):
    deg = jnp.sum(deg_ref[0], axis=0) + 1.0
    dinv = lax.rsqrt(deg)
    s = s_ref[0] + s_ref[1] + g_ref[...]
    tr = (1.0 - ALPHA) * (s * dinv[:, None]) + ALPHA * h_ref[...]
    o_ref[...] = (1.0 - MIX_B) * tr + MIX_B * jnp.dot(
        tr, w_ref[...], preferred_element_type=jnp.float32)


_final_kernel = pl.pallas_call(
    _final_body,
    grid=(N // BN,),
    in_specs=[
        pl.BlockSpec((1, NW, BN), lambda j: (j, 0, 0)),
        pl.BlockSpec((NC, BN, D), lambda j: (0, j, 0)),
        pl.BlockSpec((BN, D), lambda j: (j, 0)),
        pl.BlockSpec((BN, D), lambda j: (j, 0)),
        pl.BlockSpec((D, D), lambda j: (0, 0)),
    ],
    out_specs=pl.BlockSpec((BN, D), lambda j: (j, 0)),
    out_shape=jax.ShapeDtypeStruct((N, D), jnp.float32),
)


def kernel(features, H0, W, edge_index):
    edges = edge_index.reshape(2 * E)
    deg_p = _deg_kernel(edges)
    deg_t = deg_p.reshape(NW, N // BN, BN).transpose(1, 0, 2)
    g = _scale_kernel(deg_t, features)
    s_p = _agg_kernel(edges, g)
    return _final_kernel(deg_t, s_p, g, H0, W)


# speedup vs baseline: 1.2698x; 1.2698x over previous
import functools
import math

import jax
import jax.numpy as jnp
from jax import lax
from jax.experimental import pallas as pl
from jax.experimental.pallas import tpu as pltpu
from jax.experimental.pallas import tpu_sc as plsc

N = 10000
D = 128
E = 320000
ALPHA = 0.1
MIX_B = math.log1p(0.5 / 3.0)

NC = 2
NS = 16
NW = NC * NS
L = 16

EPW = E // NW
EPC = E // NC
RPS = N // NS
RCHUNK = 80
C = 80
CHUNKS = EPW // C
PAIRS = CHUNKS // 2
LAST = 2 * PAIRS
BN = 1000

_mesh = plsc.VectorSubcoreMesh(core_axis_name="c", subcore_axis_name="s")


@functools.partial(
    pl.kernel,
    out_type=jax.ShapeDtypeStruct((NW * N,), jnp.float32),
    mesh=_mesh,
    scratch_types=[
        pltpu.VMEM((EPW,), jnp.int32),
        pltpu.VMEM((N,), jnp.float32),
    ],
    compiler_params=pltpu.CompilerParams(needs_layout_passes=False),
)
def _deg_kernel(edges_hbm, deg_out, dstbuf, degbuf):
    cid = lax.axis_index("c")
    sid = lax.axis_index("s")
    wid = sid * NC + cid

    zeros = jnp.zeros((L,), jnp.float32)
    ones = jnp.ones((L,), jnp.float32)

    def _zero(i):
        degbuf[pl.ds(i * L, L)] = zeros

    pl.loop(0, N // L)(_zero)

    pltpu.sync_copy(edges_hbm.at[pl.ds(E + wid * EPW, EPW)], dstbuf)

    def _count(i):
        idx = dstbuf[pl.ds(i * L, L)]
        plsc.addupdate_scatter(degbuf, [idx], ones)

    pl.loop(0, EPW // L)(_count)

    pltpu.sync_copy(degbuf, deg_out.at[pl.ds(wid * N, N)])


def _scale_body(deg_ref, f_ref, g_ref):
    deg = jnp.sum(deg_ref[0], axis=0) + 1.0
    dinv = lax.rsqrt(deg)
    g_ref[...] = f_ref[...] * dinv[:, None]


_scale_kernel = pl.pallas_call(
    _scale_body,
    grid=(N // BN,),
    in_specs=[
        pl.BlockSpec((1, NW, BN), lambda j: (j, 0, 0)),
        pl.BlockSpec((BN, D), lambda j: (j, 0)),
    ],
    out_specs=pl.BlockSpec((BN, D), lambda j: (j, 0)),
    out_shape=jax.ShapeDtypeStruct((N, D), jnp.float32),
)


@functools.partial(
    pl.kernel,
    out_type=jax.ShapeDtypeStruct((NC, N, D), jnp.float32),
    mesh=_mesh,
    scratch_types=[
        pltpu.VMEM_SHARED((N, D), jnp.float32),
        pltpu.VMEM((8, C), jnp.int32),
        pltpu.VMEM((8, C), jnp.int32),
        pltpu.VMEM((4 * C, D), jnp.float32),
        pltpu.SemaphoreType.DMA((4,)),
        pltpu.SemaphoreType.DMA((4,)),
        pltpu.SemaphoreType.DMA((4,)),
        pltpu.SemaphoreType.DMA,
    ],
)
def _agg_kernel(edges_hbm, g_hbm, s_out, acc, srcb, dstb,
                rows, gsems, ssems, dsems, asem):
    cid = lax.axis_index("c")
    sid = lax.axis_index("s")
    wid = cid * NS + sid

    zv = jnp.zeros((L,), jnp.float32)

    def _zrow(r):
        def _zcol(j):
            rows[r, pl.ds(j * L, L)] = zv

        pl.loop(0, D // L)(_zcol)

    pl.loop(0, C)(_zrow)

    def _zero(c):
        pltpu.sync_copy(rows.at[pl.ds(0, C)],
                        acc.at[pl.ds(c * RCHUNK, RCHUNK)])

    pl.loop(sid, N // RCHUNK, step=NS)(_zero)

    def _idx_start(t):
        s = t % 8
        off = wid * EPW + t * C
        pltpu.async_copy(edges_hbm.at[pl.ds(off, C)], srcb.at[s],
                         ssems.at[t % 4])
        pltpu.async_copy(edges_hbm.at[pl.ds(E + off, C)], dstb.at[s],
                         dsems.at[t % 4])

    def _idx_wait(t):
        s = t % 8
        pltpu.make_async_copy(edges_hbm.at[pl.ds(0, C)], srcb.at[s],
                              ssems.at[t % 4]).wait()
        pltpu.make_async_copy(edges_hbm.at[pl.ds(0, C)], dstb.at[s],
                              dsems.at[t % 4]).wait()

    def _gather_start(t):
        pltpu.async_copy(g_hbm.at[srcb.at[t % 8]],
                         rows.at[pl.ds((t % 4) * C, C)], gsems.at[t % 4])

    def _gather_wait(t):
        pltpu.make_async_copy(g_hbm.at[pl.ds(0, C)],
                              rows.at[pl.ds((t % 4) * C, C)],
                              gsems.at[t % 4]).wait()

    def _scatter_wait(t):
        pltpu.make_async_copy(rows.at[pl.ds((t % 4) * C, C)],
                              acc.at[dstb.at[t % 8]], asem).wait()

    _idx_start(0)
    _idx_start(1)
    _idx_start(2)
    _idx_start(3)
    plsc.subcore_barrier()
    for t in range(3):
        _idx_wait(t)
        _gather_start(t)

    def _step(t):
        _gather_wait(t)

        @pl.when(t > 0)
        def _():
            _scatter_wait(t - 1)

        @pl.when(t + 3 < CHUNKS)
        def _():
            _idx_wait(t + 3)
            _gather_start(t + 3)

        @pl.when(t + 4 < CHUNKS)
        def _():
            _idx_start(t + 4)

        pltpu.async_copy(rows.at[pl.ds((t % 4) * C, C)],
                         acc.at[dstb.at[t % 8]], asem, add=True)

    pl.loop(0, CHUNKS)(_step)

    _scatter_wait(CHUNKS - 1)
    plsc.subcore_barrier()

    def _dump(c):
        pltpu.sync_copy(acc.at[pl.ds(c * RCHUNK, RCHUNK)],
                        s_out.at[cid, pl.ds(c * RCHUNK, RCHUNK)])

    pl.loop(sid, N // RCHUNK, step=NS)(_dump)


def _final_body(deg_ref, s_ref, g_ref, h_ref, w_ref, o_ref):
    deg = jnp.sum(deg_ref[0], axis=0) + 1.0
    dinv = lax.rsqrt(deg)
    s = s_ref[0] + s_ref[1] + g_ref[...]
    tr = (1.0 - ALPHA) * (s * dinv[:, None]) + ALPHA * h_ref[...]
    o_ref[...] = (1.0 - MIX_B) * tr + MIX_B * jnp.dot(
        tr, w_ref[...], preferred_element_type=jnp.float32)


_final_kernel = pl.pallas_call(
    _final_body,
    grid=(N // BN,),
    in_specs=[
        pl.BlockSpec((1, NW, BN), lambda j: (j, 0, 0)),
        pl.BlockSpec((NC, BN, D), lambda j: (0, j, 0)),
        pl.BlockSpec((BN, D), lambda j: (j, 0)),
        pl.BlockSpec((BN, D), lambda j: (j, 0)),
        pl.BlockSpec((D, D), lambda j: (0, 0)),
    ],
    out_specs=pl.BlockSpec((BN, D), lambda j: (j, 0)),
    out_shape=jax.ShapeDtypeStruct((N, D), jnp.float32),
)


def kernel(features, H0, W, edge_index):
    edges = edge_index.reshape(2 * E)
    deg_p = _deg_kernel(edges)
    deg_t = deg_p.reshape(NW, N // BN, BN).transpose(1, 0, 2)
    g = _scale_kernel(deg_t, features)
    s_p = _agg_kernel(edges, g)
    return _final_kernel(deg_t, s_p, g, H0, W)


# speedup vs baseline: 1.3044x; 1.0273x over previous
import functools
import math

import jax
import jax.numpy as jnp
from jax import lax
from jax.experimental import pallas as pl
from jax.experimental.pallas import tpu as pltpu
from jax.experimental.pallas import tpu_sc as plsc

N = 10000
D = 128
E = 320000
ALPHA = 0.1
MIX_B = math.log1p(0.5 / 3.0)

NC = 2
NS = 16
NW = NC * NS
L = 16

EPW = E // NW
EPC = E // NC
RPS = N // NS
RCHUNK = 80
C = 80
CHUNKS = EPW // C
PAIRS = CHUNKS // 2
LAST = 2 * PAIRS
BN = 1000

_mesh = plsc.VectorSubcoreMesh(core_axis_name="c", subcore_axis_name="s")


@functools.partial(
    pl.kernel,
    out_type=jax.ShapeDtypeStruct((NW * N,), jnp.float32),
    mesh=_mesh,
    scratch_types=[
        pltpu.VMEM((EPW,), jnp.int32),
        pltpu.VMEM((N,), jnp.float32),
    ],
    compiler_params=pltpu.CompilerParams(needs_layout_passes=False),
)
def _deg_kernel(edges_hbm, deg_out, dstbuf, degbuf):
    cid = lax.axis_index("c")
    sid = lax.axis_index("s")
    wid = sid * NC + cid

    zeros = jnp.zeros((L,), jnp.float32)
    ones = jnp.ones((L,), jnp.float32)

    def _zero(i):
        degbuf[pl.ds(i * L, L)] = zeros

    pl.loop(0, N // L, unroll=8)(_zero)

    pltpu.sync_copy(edges_hbm.at[pl.ds(E + wid * EPW, EPW)], dstbuf)

    def _count(i):
        idx = dstbuf[pl.ds(i * L, L)]
        plsc.addupdate_scatter(degbuf, [idx], ones)

    pl.loop(0, EPW // L, unroll=8)(_count)

    pltpu.sync_copy(degbuf, deg_out.at[pl.ds(wid * N, N)])


def _scale_body(deg_ref, f_ref, g_ref):
    deg = jnp.sum(deg_ref[0], axis=0) + 1.0
    dinv = lax.rsqrt(deg)
    g_ref[...] = f_ref[...] * dinv[:, None]


_scale_kernel = pl.pallas_call(
    _scale_body,
    grid=(N // BN,),
    in_specs=[
        pl.BlockSpec((1, NW, BN), lambda j: (j, 0, 0)),
        pl.BlockSpec((BN, D), lambda j: (j, 0)),
    ],
    out_specs=pl.BlockSpec((BN, D), lambda j: (j, 0)),
    out_shape=jax.ShapeDtypeStruct((N, D), jnp.float32),
)


@functools.partial(
    pl.kernel,
    out_type=jax.ShapeDtypeStruct((NC, N, D), jnp.float32),
    mesh=_mesh,
    scratch_types=[
        pltpu.VMEM_SHARED((N, D), jnp.float32),
        pltpu.VMEM((8, C), jnp.int32),
        pltpu.VMEM((8, C), jnp.int32),
        pltpu.VMEM((4 * C, D), jnp.float32),
        pltpu.SemaphoreType.DMA((4,)),
        pltpu.SemaphoreType.DMA((4,)),
        pltpu.SemaphoreType.DMA((4,)),
        pltpu.SemaphoreType.DMA,
    ],
)
def _agg_kernel(edges_hbm, g_hbm, s_out, acc, srcb, dstb,
                rows, gsems, ssems, dsems, asem):
    cid = lax.axis_index("c")
    sid = lax.axis_index("s")
    wid = cid * NS + sid

    zv = jnp.zeros((L,), jnp.float32)

    def _zrow(r):
        def _zcol(j):
            rows[r, pl.ds(j * L, L)] = zv

        pl.loop(0, D // L)(_zcol)

    pl.loop(3 * C, 4 * C, unroll=4)(_zrow)

    def _idx_start(t):
        s = t % 8
        off = wid * EPW + t * C
        pltpu.async_copy(edges_hbm.at[pl.ds(off, C)], srcb.at[s],
                         ssems.at[t % 4])
        pltpu.async_copy(edges_hbm.at[pl.ds(E + off, C)], dstb.at[s],
                         dsems.at[t % 4])

    def _idx_wait(t):
        s = t % 8
        pltpu.make_async_copy(edges_hbm.at[pl.ds(0, C)], srcb.at[s],
                              ssems.at[t % 4]).wait()
        pltpu.make_async_copy(edges_hbm.at[pl.ds(0, C)], dstb.at[s],
                              dsems.at[t % 4]).wait()

    def _gather_start(t):
        pltpu.async_copy(g_hbm.at[srcb.at[t % 8]],
                         rows.at[pl.ds((t % 4) * C, C)], gsems.at[t % 4])

    def _gather_wait(t):
        pltpu.make_async_copy(g_hbm.at[pl.ds(0, C)],
                              rows.at[pl.ds((t % 4) * C, C)],
                              gsems.at[t % 4]).wait()

    def _scatter_wait(t):
        pltpu.make_async_copy(rows.at[pl.ds((t % 4) * C, C)],
                              acc.at[dstb.at[t % 8]], asem).wait()

    _idx_start(0)
    _idx_start(1)
    _idx_start(2)
    _idx_start(3)
    for t in range(3):
        _idx_wait(t)
        _gather_start(t)

    def _zero(c):
        pltpu.sync_copy(rows.at[pl.ds(3 * C, C)],
                        acc.at[pl.ds(c * RCHUNK, RCHUNK)])

    pl.loop(sid, N // RCHUNK, step=NS)(_zero)
    plsc.subcore_barrier()

    def _step(t):
        _gather_wait(t)

        @pl.when(t > 0)
        def _():
            _scatter_wait(t - 1)

        @pl.when(t + 3 < CHUNKS)
        def _():
            _idx_wait(t + 3)
            _gather_start(t + 3)

        @pl.when(t + 4 < CHUNKS)
        def _():
            _idx_start(t + 4)

        pltpu.async_copy(rows.at[pl.ds((t % 4) * C, C)],
                         acc.at[dstb.at[t % 8]], asem, add=True)

    pl.loop(0, CHUNKS)(_step)

    _scatter_wait(CHUNKS - 1)
    plsc.subcore_barrier()

    def _dump(c):
        pltpu.async_copy(acc.at[pl.ds(c * RCHUNK, RCHUNK)],
                         s_out.at[cid, pl.ds(c * RCHUNK, RCHUNK)], asem)

    pl.loop(sid, N // RCHUNK, step=NS)(_dump)

    def _dump_wait(c):
        pltpu.make_async_copy(acc.at[pl.ds(0, RCHUNK)],
                              s_out.at[cid, pl.ds(0, RCHUNK)], asem).wait()

    pl.loop(sid, N // RCHUNK, step=NS)(_dump_wait)


def _final_body(deg_ref, s_ref, g_ref, h_ref, w_ref, o_ref):
    deg = jnp.sum(deg_ref[0], axis=0) + 1.0
    dinv = lax.rsqrt(deg)
    s = s_ref[0] + s_ref[1] + g_ref[...]
    tr = (1.0 - ALPHA) * (s * dinv[:, None]) + ALPHA * h_ref[...]
    o_ref[...] = (1.0 - MIX_B) * tr + MIX_B * jnp.dot(
        tr, w_ref[...], preferred_element_type=jnp.float32)


_final_kernel = pl.pallas_call(
    _final_body,
    grid=(N // BN,),
    in_specs=[
        pl.BlockSpec((1, NW, BN), lambda j: (j, 0, 0)),
        pl.BlockSpec((NC, BN, D), lambda j: (0, j, 0)),
        pl.BlockSpec((BN, D), lambda j: (j, 0)),
        pl.BlockSpec((BN, D), lambda j: (j, 0)),
        pl.BlockSpec((D, D), lambda j: (0, 0)),
    ],
    out_specs=pl.BlockSpec((BN, D), lambda j: (j, 0)),
    out_shape=jax.ShapeDtypeStruct((N, D), jnp.float32),
)


def kernel(features, H0, W, edge_index):
    edges = edge_index.reshape(2 * E)
    deg_p = _deg_kernel(edges)
    deg_t = deg_p.reshape(NW, N // BN, BN).transpose(1, 0, 2)
    g = _scale_kernel(deg_t, features)
    s_p = _agg_kernel(edges, g)
    return _final_kernel(deg_t, s_p, g, H0, W)


# speedup vs baseline: 1.3051x; 1.0005x over previous
import functools
import math

import jax
import jax.numpy as jnp
from jax import lax
from jax.experimental import pallas as pl
from jax.experimental.pallas import tpu as pltpu
from jax.experimental.pallas import tpu_sc as plsc

N = 10000
D = 128
E = 320000
ALPHA = 0.1
MIX_B = math.log1p(0.5 / 3.0)

NC = 2
NS = 16
NW = NC * NS
L = 16

EPW = E // NW
RCHUNK = 80
C = 80
CHUNKS = EPW // C
BN = 1000

_mesh = plsc.VectorSubcoreMesh(core_axis_name="c", subcore_axis_name="s")


@functools.partial(
    pl.kernel,
    out_type=jax.ShapeDtypeStruct((NW * N,), jnp.float32),
    mesh=_mesh,
    scratch_types=[
        pltpu.VMEM((EPW,), jnp.int32),
        pltpu.VMEM((N,), jnp.float32),
    ],
    compiler_params=pltpu.CompilerParams(needs_layout_passes=False),
)
def _deg_kernel(edges_hbm, deg_out, dstbuf, degbuf):
    cid = lax.axis_index("c")
    sid = lax.axis_index("s")
    wid = sid * NC + cid

    zeros = jnp.zeros((L,), jnp.float32)
    ones = jnp.ones((L,), jnp.float32)

    def _zero(i):
        degbuf[pl.ds(i * L, L)] = zeros

    pl.loop(0, N // L, unroll=8)(_zero)

    pltpu.sync_copy(edges_hbm.at[pl.ds(E + wid * EPW, EPW)], dstbuf)

    def _count(i):
        idx = dstbuf[pl.ds(i * L, L)]
        plsc.addupdate_scatter(degbuf, [idx], ones)

    pl.loop(0, EPW // L, unroll=8)(_count)

    pltpu.sync_copy(degbuf, deg_out.at[pl.ds(wid * N, N)])


def _scale_body(deg_ref, f_ref, g_ref):
    deg = jnp.sum(deg_ref[0], axis=0) + 1.0
    dinv = lax.rsqrt(deg)
    g_ref[...] = f_ref[...] * dinv[:, None]


_scale_kernel = pl.pallas_call(
    _scale_body,
    grid=(N // BN,),
    in_specs=[
        pl.BlockSpec((1, NW, BN), lambda j: (j, 0, 0)),
        pl.BlockSpec((BN, D), lambda j: (j, 0)),
    ],
    out_specs=pl.BlockSpec((BN, D), lambda j: (j, 0)),
    out_shape=jax.ShapeDtypeStruct((N, D), jnp.float32),
)


@functools.partial(
    pl.kernel,
    out_type=jax.ShapeDtypeStruct((NC, N, D), jnp.float32),
    mesh=_mesh,
    scratch_types=[
        pltpu.VMEM_SHARED((N, D), jnp.float32),
        pltpu.VMEM((8, C), jnp.int32),
        pltpu.VMEM((8, C), jnp.int32),
        pltpu.VMEM((4 * C, D), jnp.float32),
        pltpu.SemaphoreType.DMA((4,)),
        pltpu.SemaphoreType.DMA((4,)),
        pltpu.SemaphoreType.DMA((4,)),
        pltpu.SemaphoreType.DMA,
    ],
)
def _agg_kernel(edges_hbm, g_hbm, s_out, acc, srcb, dstb,
                rows, gsems, ssems, dsems, asem):
    cid = lax.axis_index("c")
    sid = lax.axis_index("s")
    wid = cid * NS + sid

    zv = jnp.zeros((L,), jnp.float32)

    def _zrow(r):
        def _zcol(j):
            rows[r, pl.ds(j * L, L)] = zv

        pl.loop(0, D // L)(_zcol)

    pl.loop(3 * C, 4 * C, unroll=4)(_zrow)

    def _idx_start(t):
        s = t % 8
        off = wid * EPW + t * C
        pltpu.async_copy(edges_hbm.at[pl.ds(off, C)], srcb.at[s],
                         ssems.at[t % 4])
        pltpu.async_copy(edges_hbm.at[pl.ds(E + off, C)], dstb.at[s],
                         dsems.at[t % 4])

    def _idx_wait(t):
        s = t % 8
        pltpu.make_async_copy(edges_hbm.at[pl.ds(0, C)], srcb.at[s],
                              ssems.at[t % 4]).wait()
        pltpu.make_async_copy(edges_hbm.at[pl.ds(0, C)], dstb.at[s],
                              dsems.at[t % 4]).wait()

    def _gather_start(t):
        pltpu.async_copy(g_hbm.at[srcb.at[t % 8]],
                         rows.at[pl.ds((t % 4) * C, C)], gsems.at[t % 4])

    def _gather_wait(t):
        pltpu.make_async_copy(g_hbm.at[pl.ds(0, C)],
                              rows.at[pl.ds((t % 4) * C, C)],
                              gsems.at[t % 4]).wait()

    def _scatter_wait(t):
        pltpu.make_async_copy(rows.at[pl.ds((t % 4) * C, C)],
                              acc.at[dstb.at[t % 8]], asem).wait()

    _idx_start(0)
    _idx_start(1)
    _idx_start(2)
    _idx_start(3)
    for t in range(3):
        _idx_wait(t)
        _gather_start(t)

    def _zero(c):
        pltpu.sync_copy(rows.at[pl.ds(3 * C, C)],
                        acc.at[pl.ds(c * RCHUNK, RCHUNK)])

    pl.loop(sid, N // RCHUNK, step=NS)(_zero)
    plsc.subcore_barrier()

    def _step(t):
        _gather_wait(t)

        @pl.when(t > 0)
        def _():
            _scatter_wait(t - 1)

        @pl.when(t + 3 < CHUNKS)
        def _():
            _idx_wait(t + 3)
            _gather_start(t + 3)

        @pl.when(t + 4 < CHUNKS)
        def _():
            _idx_start(t + 4)

        pltpu.async_copy(rows.at[pl.ds((t % 4) * C, C)],
                         acc.at[dstb.at[t % 8]], asem, add=True)

    pl.loop(0, CHUNKS)(_step)

    _scatter_wait(CHUNKS - 1)
    plsc.subcore_barrier()

    def _dump(c):
        pltpu.async_copy(acc.at[pl.ds(c * RCHUNK, RCHUNK)],
                         s_out.at[cid, pl.ds(c * RCHUNK, RCHUNK)], asem)

    pl.loop(sid, N // RCHUNK, step=NS)(_dump)

    def _dump_wait(c):
        pltpu.make_async_copy(acc.at[pl.ds(0, RCHUNK)],
                              s_out.at[cid, pl.ds(0, RCHUNK)], asem).wait()

    pl.loop(sid, N // RCHUNK, step=NS)(_dump_wait)


def _final_body(deg_ref, s_ref, g_ref, h_ref, w_ref, o_ref):
    deg = jnp.sum(deg_ref[0], axis=0) + 1.0
    dinv = lax.rsqrt(deg)
    s = s_ref[0] + s_ref[1] + g_ref[...]
    tr = (1.0 - ALPHA) * (s * dinv[:, None]) + ALPHA * h_ref[...]
    o_ref[...] = (1.0 - MIX_B) * tr + MIX_B * jnp.dot(
        tr, w_ref[...], preferred_element_type=jnp.float32)


_final_kernel = pl.pallas_call(
    _final_body,
    grid=(N // BN,),
    in_specs=[
        pl.BlockSpec((1, NW, BN), lambda j: (j, 0, 0)),
        pl.BlockSpec((NC, BN, D), lambda j: (0, j, 0)),
        pl.BlockSpec((BN, D), lambda j: (j, 0)),
        pl.BlockSpec((BN, D), lambda j: (j, 0)),
        pl.BlockSpec((D, D), lambda j: (0, 0)),
    ],
    out_specs=pl.BlockSpec((BN, D), lambda j: (j, 0)),
    out_shape=jax.ShapeDtypeStruct((N, D), jnp.float32),
)


def kernel(features, H0, W, edge_index):
    edges = edge_index.reshape(2 * E)
    deg_p = _deg_kernel(edges)
    deg_t = deg_p.reshape(NW, N // BN, BN).transpose(1, 0, 2)
    g = _scale_kernel(deg_t, features)
    s_p = _agg_kernel(edges, g)
    return _final_kernel(deg_t, s_p, g, H0, W)
